# baseline (speedup 1.0000x reference)
import jax
import jax.numpy as jnp
from jax.experimental import pallas as pl

_HI = jax.lax.Precision.HIGHEST


def _knn_idx_hi(x, k):
    sq = jnp.sum(x * x, axis=1)
    hi = x.astype(jnp.bfloat16)
    lo = (x - hi.astype(jnp.float32)).astype(jnp.bfloat16)
    f32 = jnp.float32
    g = (jnp.dot(hi, hi.T, precision=_HI, preferred_element_type=f32)
         + jnp.dot(hi, lo.T, precision=_HI, preferred_element_type=f32)
         + jnp.dot(lo, hi.T, precision=_HI, preferred_element_type=f32))
    d = sq[:, None] + sq[None, :] - 2.0 * g
    _, idx = jax.lax.top_k(-d, k)
    return idx


def _edge_conv_hi(x, idx, W1, b1, W2, b2):
    xj = x[idx]
    xi = jnp.broadcast_to(x[:, None, :], xj.shape)
    msg = jnp.concatenate([xi, xj - xi], axis=-1)
    h = jax.nn.relu(msg @ W1 + b1)
    h = h @ W2 + b2
    return jnp.max(h, axis=1)


def kernel(x, W1, b1, W2, b2, W3, b3, W4, b4):
    idx1 = _knn_idx_hi(x, 16)
    x1 = _edge_conv_hi(x, idx1, W1, b1, W2, b2)
    idx2 = _knn_idx_hi(x1, 16)
    x2 = _edge_conv_hi(x1, idx2, W3, b3, W4, b4)
    return x2


# speedup vs baseline: 3.7003x; 3.7003x over previous
import functools

import jax
import jax.numpy as jnp
from jax import lax
from jax.experimental import pallas as pl
from jax.experimental.pallas import tpu as pltpu
from jax.experimental.pallas import tpu_sc as plsc

NPTS = 10000
NPAD = 10240
K = 16
BR = 256
BR2 = 512
F32 = jnp.float32
BF16 = jnp.bfloat16


def _knn_body(hi_r, lo_r, hi_ct, lo_ct, sq_r, sq_c, idx_out, d_ref, it_ref):
    dn = (((1,), (0,)), ((), ()))
    g = lax.dot_general(hi_r[...], hi_ct[...], dn, preferred_element_type=F32)
    g = g + lax.dot_general(hi_r[...], lo_ct[...], dn, preferred_element_type=F32)
    g = g + lax.dot_general(lo_r[...], hi_ct[...], dn, preferred_element_type=F32)
    d_ref[...] = sq_r[...] + sq_c[...] - 2.0 * g
    it_ref[...] = lax.broadcasted_iota(jnp.int32, (BR, NPAD), 1)

    def round_body(t, _):
        d = d_ref[...]
        iota = it_ref[...]
        m = jnp.min(d, axis=1, keepdims=True)
        eq = d == m
        jf = jnp.min(jnp.where(eq, iota, jnp.int32(2**31 - 1)), axis=1,
                     keepdims=True)
        d_ref[...] = jnp.where(eq & (iota == jf), jnp.inf, d)
        idx_out[pl.ds(t, 1), :] = jf.reshape(1, BR)
        return 0

    lax.fori_loop(0, K, round_body, 0)


def _knn_topk(x_pad, sq_pad):
    hi = x_pad.astype(BF16)
    lo = (x_pad - hi.astype(F32)).astype(BF16)
    hi_t = hi.T
    lo_t = lo.T
    sq_r = sq_pad.reshape(NPAD, 1)
    sq_c = sq_pad.reshape(1, NPAD)
    d = x_pad.shape[1]
    grid = NPAD // BR
    return pl.pallas_call(
        _knn_body,
        grid=(grid,),
        in_specs=[
            pl.BlockSpec((BR, d), lambda i: (i, 0)),
            pl.BlockSpec((BR, d), lambda i: (i, 0)),
            pl.BlockSpec((d, NPAD), lambda i: (0, 0)),
            pl.BlockSpec((d, NPAD), lambda i: (0, 0)),
            pl.BlockSpec((BR, 1), lambda i: (i, 0)),
            pl.BlockSpec((1, NPAD), lambda i: (0, 0)),
        ],
        out_specs=pl.BlockSpec((K, BR), lambda i: (0, i)),
        out_shape=jax.ShapeDtypeStruct((K, NPAD), jnp.int32),
        scratch_shapes=[
            pltpu.VMEM((BR, NPAD), F32),
            pltpu.VMEM((BR, NPAD), jnp.int32),
        ],
    )(hi, lo, hi_t, lo_t, sq_r, sq_c)


_NC = 2
_NS = 16
_NW = _NC * _NS
_CH = 128
_NCHUNK = (NPAD * K) // (_NW * _CH)


def _sc_gather(table, idx_rows):
    h = table.shape[1]
    mesh = plsc.VectorSubcoreMesh(core_axis_name="c", subcore_axis_name="s")

    @functools.partial(
        pl.kernel,
        mesh=mesh,
        compiler_params=pltpu.CompilerParams(use_tc_tiling_on_sc=False),
        out_type=jax.ShapeDtypeStruct((NPAD * K, h), F32),
        scratch_types=[
            pltpu.VMEM((_NCHUNK, _CH), jnp.int32),
            pltpu.VMEM((_CH, h), F32),
            pltpu.SemaphoreType.DMA,
        ],
    )
    def gather_kernel(table_hbm, idx_hbm, out_hbm, idx_v, rows_v, sem):
        wid = lax.axis_index("s") * _NC + lax.axis_index("c")
        pltpu.sync_copy(idx_hbm.at[pl.ds(wid * _NCHUNK, _NCHUNK)], idx_v)
        base = wid * (_NCHUNK * _CH)

        def chunk(j):
            pltpu.async_copy(table_hbm.at[idx_v.at[j]], rows_v, sem).wait()
            pltpu.sync_copy(rows_v, out_hbm.at[pl.ds(base + j * _CH, _CH)])

        pl.loop(0, _NCHUNK)(chunk)

    return gather_kernel(table, idx_rows)


def _edge_body(xi_ref, xg_ref, w1t_ref, w1b_ref, b1_ref, w2_ref, b2_ref,
               out_ref):
    dp = xi_ref.shape[1]
    dn = (((1,), (0,)), ((), ()))
    xi = xi_ref[...]
    xj = xg_ref[...].reshape(BR2, K, dp)
    diff = (xj - xi[:, None, :]).reshape(BR2 * K, dp)
    xi_rep = jnp.broadcast_to(xi[:, None, :], (BR2, K, dp)).reshape(BR2 * K, dp)
    h1 = lax.dot_general(xi_rep.astype(BF16), w1t_ref[...], dn,
                         preferred_element_type=F32)
    h1 = h1 + lax.dot_general(diff.astype(BF16), w1b_ref[...], dn,
                              preferred_element_type=F32)
    h1 = jnp.maximum(h1 + b1_ref[...], 0.0)
    h2 = lax.dot_general(h1.astype(BF16), w2_ref[...], dn,
                         preferred_element_type=F32)
    h2 = h2 + b2_ref[...]
    out_ref[...] = jnp.max(h2.reshape(BR2, K, -1), axis=1)


def _edge_mlp_max(xi_p, xg, w1t, w1b, b1, w2, b2):
    dp = xi_p.shape[1]
    o = w2.shape[1]
    grid = NPAD // BR2
    return pl.pallas_call(
        _edge_body,
        grid=(grid,),
        in_specs=[
            pl.BlockSpec((BR2, dp), lambda i: (i, 0)),
            pl.BlockSpec((BR2 * K, dp), lambda i: (i, 0)),
            pl.BlockSpec((dp, o), lambda i: (0, 0)),
            pl.BlockSpec((dp, o), lambda i: (0, 0)),
            pl.BlockSpec((1, o), lambda i: (0, 0)),
            pl.BlockSpec((o, o), lambda i: (0, 0)),
            pl.BlockSpec((1, o), lambda i: (0, 0)),
        ],
        out_specs=pl.BlockSpec((BR2, o), lambda i: (i, 0)),
        out_shape=jax.ShapeDtypeStruct((NPAD, o), F32),
    )(xi_p, xg, w1t, w1b, b1.reshape(1, o), w2, b2.reshape(1, o))


def _layer(x_pad, w1, b1, w2, b2):
    d = x_pad.shape[1]
    dp = max(16, d)
    sq = jnp.sum(x_pad * x_pad, axis=1)
    sq = jnp.where(jnp.arange(NPAD) < NPTS, sq, jnp.inf)
    idx_t = _knn_topk(x_pad, sq)
    xi_p = jnp.pad(x_pad, ((0, 0), (0, dp - d)))
    w1t = jnp.pad(w1[:d], ((0, dp - d), (0, 0))).astype(BF16)
    w1b = jnp.pad(w1[d:], ((0, dp - d), (0, 0))).astype(BF16)
    idx_rows = idx_t.T.reshape(_NW * _NCHUNK, _CH)
    xg = _sc_gather(xi_p, idx_rows)
    return _edge_mlp_max(xi_p, xg, w1t, w1b, b1, w2.astype(BF16), b2)


def kernel(x, W1, b1, W2, b2, W3, b3, W4, b4):
    x_pad = jnp.pad(x, ((0, NPAD - NPTS), (0, 0)))
    x1 = _layer(x_pad, W1, b1, W2, b2)
    x2 = _layer(x1, W3, b3, W4, b4)
    return x2[:NPTS]


# speedup vs baseline: 5.2634x; 1.4224x over previous
import functools

import jax
import jax.numpy as jnp
from jax import lax
from jax.experimental import pallas as pl
from jax.experimental.pallas import tpu as pltpu
from jax.experimental.pallas import tpu_sc as plsc

NPTS = 10000
NPAD = 10240
K = 16
BR = 512
BR2 = 512
F32 = jnp.float32
BF16 = jnp.bfloat16


def _knn_body(hi_r, lo_r, hi_ct, lo_ct, sq_r, sq_c, idx_out, d_ref):
    dn = (((1,), (0,)), ((), ()))
    g = lax.dot_general(hi_r[...], hi_ct[...], dn, preferred_element_type=F32)
    g = g + lax.dot_general(hi_r[...], lo_ct[...], dn, preferred_element_type=F32)
    g = g + lax.dot_general(lo_r[...], hi_ct[...], dn, preferred_element_type=F32)
    d_ref[...] = sq_r[...] + sq_c[...] - 2.0 * g

    def round_body(t, _):
        iota = lax.broadcasted_iota(jnp.int32, (BR, NPAD), 1)
        m = jnp.min(d_ref[...], axis=1, keepdims=True)
        jf = jnp.min(jnp.where(d_ref[...] == m, iota,
                               jnp.int32(2**31 - 1)), axis=1, keepdims=True)
        d_ref[...] = jnp.where(iota == jf, jnp.inf, d_ref[...])
        idx_out[pl.ds(t, 1), :] = jf.reshape(1, BR)
        return 0

    lax.fori_loop(0, K, round_body, 0)


def _knn_topk(x_pad, sq_pad):
    hi = x_pad.astype(BF16)
    lo = (x_pad - hi.astype(F32)).astype(BF16)
    hi_t = hi.T
    lo_t = lo.T
    sq_r = sq_pad.reshape(NPAD, 1)
    sq_c = sq_pad.reshape(1, NPAD)
    d = x_pad.shape[1]
    grid = NPAD // BR
    return pl.pallas_call(
        _knn_body,
        grid=(grid,),
        in_specs=[
            pl.BlockSpec((BR, d), lambda i: (i, 0)),
            pl.BlockSpec((BR, d), lambda i: (i, 0)),
            pl.BlockSpec((d, NPAD), lambda i: (0, 0)),
            pl.BlockSpec((d, NPAD), lambda i: (0, 0)),
            pl.BlockSpec((BR, 1), lambda i: (i, 0)),
            pl.BlockSpec((1, NPAD), lambda i: (0, 0)),
        ],
        out_specs=pl.BlockSpec((K, BR), lambda i: (0, i)),
        out_shape=jax.ShapeDtypeStruct((K, NPAD), jnp.int32),
        scratch_shapes=[
            pltpu.VMEM((BR, NPAD), F32),
        ],
    )(hi, lo, hi_t, lo_t, sq_r, sq_c)


_NC = 2
_NS = 16
_NW = _NC * _NS
_CH = 128
_NCHUNK = (NPAD * K) // (_NW * _CH)


def _sc_gather(table, idx_rows):
    h = table.shape[1]
    mesh = plsc.VectorSubcoreMesh(core_axis_name="c", subcore_axis_name="s")

    @functools.partial(
        pl.kernel,
        mesh=mesh,
        compiler_params=pltpu.CompilerParams(use_tc_tiling_on_sc=False),
        out_type=jax.ShapeDtypeStruct((NPAD * K, h), F32),
        scratch_types=[
            pltpu.VMEM((_NCHUNK, _CH), jnp.int32),
            pltpu.VMEM((_CH, h), F32),
            pltpu.SemaphoreType.DMA,
        ],
    )
    def gather_kernel(table_hbm, idx_hbm, out_hbm, idx_v, rows_v, sem):
        wid = lax.axis_index("s") * _NC + lax.axis_index("c")
        pltpu.sync_copy(idx_hbm.at[pl.ds(wid * _NCHUNK, _NCHUNK)], idx_v)
        base = wid * (_NCHUNK * _CH)

        def chunk(j):
            pltpu.async_copy(table_hbm.at[idx_v.at[j]], rows_v, sem).wait()
            pltpu.sync_copy(rows_v, out_hbm.at[pl.ds(base + j * _CH, _CH)])

        pl.loop(0, _NCHUNK)(chunk)

    return gather_kernel(table, idx_rows)


def _edge_body(xi_ref, xg_ref, w1t_ref, w1b_ref, b1_ref, w2_ref, b2_ref,
               out_ref):
    dp = xi_ref.shape[1]
    dn = (((1,), (0,)), ((), ()))
    xi = xi_ref[...]
    xj = xg_ref[...].reshape(BR2, K, dp)
    diff = (xj - xi[:, None, :]).reshape(BR2 * K, dp)
    xi_rep = jnp.broadcast_to(xi[:, None, :], (BR2, K, dp)).reshape(BR2 * K, dp)
    h1 = lax.dot_general(xi_rep.astype(BF16), w1t_ref[...], dn,
                         preferred_element_type=F32)
    h1 = h1 + lax.dot_general(diff.astype(BF16), w1b_ref[...], dn,
                              preferred_element_type=F32)
    h1 = jnp.maximum(h1 + b1_ref[...], 0.0)
    h2 = lax.dot_general(h1.astype(BF16), w2_ref[...], dn,
                         preferred_element_type=F32)
    h2 = h2 + b2_ref[...]
    out_ref[...] = jnp.max(h2.reshape(BR2, K, -1), axis=1)


def _edge_mlp_max(xi_p, xg, w1t, w1b, b1, w2, b2):
    dp = xi_p.shape[1]
    o = w2.shape[1]
    grid = NPAD // BR2
    return pl.pallas_call(
        _edge_body,
        grid=(grid,),
        in_specs=[
            pl.BlockSpec((BR2, dp), lambda i: (i, 0)),
            pl.BlockSpec((BR2 * K, dp), lambda i: (i, 0)),
            pl.BlockSpec((dp, o), lambda i: (0, 0)),
            pl.BlockSpec((dp, o), lambda i: (0, 0)),
            pl.BlockSpec((1, o), lambda i: (0, 0)),
            pl.BlockSpec((o, o), lambda i: (0, 0)),
            pl.BlockSpec((1, o), lambda i: (0, 0)),
        ],
        out_specs=pl.BlockSpec((BR2, o), lambda i: (i, 0)),
        out_shape=jax.ShapeDtypeStruct((NPAD, o), F32),
    )(xi_p, xg, w1t, w1b, b1.reshape(1, o), w2, b2.reshape(1, o))


def _layer(x_pad, w1, b1, w2, b2):
    d = x_pad.shape[1]
    dp = max(16, d)
    sq = jnp.sum(x_pad * x_pad, axis=1)
    sq = jnp.where(jnp.arange(NPAD) < NPTS, sq, jnp.inf)
    idx_t = _knn_topk(x_pad, sq)
    xi_p = jnp.pad(x_pad, ((0, 0), (0, dp - d)))
    w1t = jnp.pad(w1[:d], ((0, dp - d), (0, 0))).astype(BF16)
    w1b = jnp.pad(w1[d:], ((0, dp - d), (0, 0))).astype(BF16)
    idx_rows = idx_t.T.reshape(_NW * _NCHUNK, _CH)
    xg = _sc_gather(xi_p, idx_rows)
    return _edge_mlp_max(xi_p, xg, w1t, w1b, b1, w2.astype(BF16), b2)


def kernel(x, W1, b1, W2, b2, W3, b3, W4, b4):
    x_pad = jnp.pad(x, ((0, NPAD - NPTS), (0, 0)))
    x1 = _layer(x_pad, W1, b1, W2, b2)
    x2 = _layer(x1, W3, b3, W4, b4)
    return x2[:NPTS]
